# baseline (speedup 1.0000x reference)
import functools

import jax
import jax.numpy as jnp
from jax import lax
from jax.experimental import pallas as pl
from jax.experimental.pallas import tpu as pltpu
from jax.experimental.pallas import tpu_sc as plsc


@functools.lru_cache(maxsize=None)
def _build_gather3(B, D, U, I):
    info = plsc.get_sparse_core_info()
    NC, NS = info.num_cores, info.num_subcores
    NW = NC * NS
    BPW = B // NW
    CH = 128
    NCH = BPW // CH

    mesh = plsc.VectorSubcoreMesh(core_axis_name="c", subcore_axis_name="s")
    f32 = jnp.float32

    @functools.partial(
        pl.kernel,
        mesh=mesh,
        compiler_params=pltpu.CompilerParams(use_tc_tiling_on_sc=False),
        out_type=(
            jax.ShapeDtypeStruct((B, D), f32),
            jax.ShapeDtypeStruct((B, D), f32),
            jax.ShapeDtypeStruct((B, D), f32),
        ),
        scratch_types=[
            pltpu.VMEM((NCH, CH), jnp.int32),
            pltpu.VMEM((NCH, CH), jnp.int32),
            pltpu.VMEM((NCH, CH), jnp.int32),
            pltpu.VMEM((BPW, D), f32),
            pltpu.VMEM((BPW, D), f32),
            pltpu.VMEM((BPW, D), f32),
            pltpu.SemaphoreType.DMA,
            pltpu.SemaphoreType.DMA,
        ],
    )
    def gather3(uid_hbm, pid_hbm, nid_hbm, utab_hbm, itab_hbm,
                uout_hbm, pout_hbm, nout_hbm,
                uidx, pidx, nidx, urows, prows, nrows, idsem, gsem):
        wid = lax.axis_index("s") * NC + lax.axis_index("c")
        base = wid * BPW

        idc = []
        for j in range(NCH):
            off = base + j * CH
            idc.append(pltpu.async_copy(uid_hbm.at[pl.ds(off, CH)], uidx.at[j], idsem))
            idc.append(pltpu.async_copy(pid_hbm.at[pl.ds(off, CH)], pidx.at[j], idsem))
            idc.append(pltpu.async_copy(nid_hbm.at[pl.ds(off, CH)], nidx.at[j], idsem))
        for c in idc:
            c.wait()

        gc = []
        for j in range(NCH):
            sl = pl.ds(j * CH, CH)
            gc.append(pltpu.async_copy(utab_hbm.at[uidx.at[j]], urows.at[sl], gsem))
            gc.append(pltpu.async_copy(itab_hbm.at[pidx.at[j]], prows.at[sl], gsem))
            gc.append(pltpu.async_copy(itab_hbm.at[nidx.at[j]], nrows.at[sl], gsem))
        for c in gc:
            c.wait()

        out_sl = pl.ds(base, BPW)
        pltpu.sync_copy(urows, uout_hbm.at[out_sl])
        pltpu.sync_copy(prows, pout_hbm.at[out_sl])
        pltpu.sync_copy(nrows, nout_hbm.at[out_sl])

    return gather3


@functools.lru_cache(maxsize=None)
def _build_mlp(B, D, H, H2, blk):
    NB = B // blk
    cdims = (((1,), (1,)), ((), ()))

    def body(u_ref, p_ref, n_ref, w1_ref, b1_ref, w2_ref, b2_ref, w3_ref, b3_ref,
             loss_ref, score_ref, acc_ref):
        i = pl.program_id(0)
        u = u_ref[...]
        p = p_ref[...]
        n = n_ref[...]

        pos = jnp.sum(u * p, axis=1)
        neg = jnp.sum(u * n, axis=1)
        d = pos - neg
        ls = jnp.minimum(d, 0.0) - jnp.log1p(jnp.exp(-jnp.abs(d)))
        part = jnp.sum(ls)

        @pl.when(i == 0)
        def _():
            acc_ref[0] = 0.0

        acc_ref[0] += part

        w1 = w1_ref[...]
        h1 = lax.dot_general(u, w1[:, :D], cdims, preferred_element_type=jnp.float32)
        h1 = h1 + lax.dot_general(p, w1[:, D:], cdims, preferred_element_type=jnp.float32)
        h1 = jnp.maximum(h1 + b1_ref[...], 0.0)
        h2 = lax.dot_general(h1, w2_ref[...], cdims, preferred_element_type=jnp.float32)
        h2 = jnp.maximum(h2 + b2_ref[...], 0.0)
        s = jnp.sum(h2 * w3_ref[...], axis=1, keepdims=True)
        score_ref[...] = s + b3_ref[0, 0]

        @pl.when(i == NB - 1)
        def _():
            loss_ref[0, 0] = -acc_ref[0] / B

    return pl.pallas_call(
        body,
        grid=(NB,),
        in_specs=[
            pl.BlockSpec((blk, D), lambda i: (i, 0)),
            pl.BlockSpec((blk, D), lambda i: (i, 0)),
            pl.BlockSpec((blk, D), lambda i: (i, 0)),
            pl.BlockSpec((H, 2 * D), lambda i: (0, 0)),
            pl.BlockSpec((1, H), lambda i: (0, 0)),
            pl.BlockSpec((H2, H), lambda i: (0, 0)),
            pl.BlockSpec((1, H2), lambda i: (0, 0)),
            pl.BlockSpec((1, H2), lambda i: (0, 0)),
            pl.BlockSpec(memory_space=pltpu.SMEM),
        ],
        out_specs=[
            pl.BlockSpec(memory_space=pltpu.SMEM),
            pl.BlockSpec((blk, 1), lambda i: (i, 0)),
        ],
        out_shape=[
            jax.ShapeDtypeStruct((1, 1), jnp.float32),
            jax.ShapeDtypeStruct((B, 1), jnp.float32),
        ],
        scratch_shapes=[pltpu.SMEM((1,), jnp.float32)],
    )


def kernel(user_ids, pos_item_ids, neg_item_ids, user_table, item_table,
           W1, b1, W2, b2, W3, b3):
    B = user_ids.shape[0]
    U, D = user_table.shape
    I = item_table.shape[0]
    H = W1.shape[0]
    H2 = W2.shape[0]

    uids = user_ids.astype(jnp.int32)
    pids = pos_item_ids.astype(jnp.int32)
    nids = neg_item_ids.astype(jnp.int32)

    u, p, n = _build_gather3(B, D, U, I)(uids, pids, nids, user_table, item_table)

    loss, score = _build_mlp(B, D, H, H2, 1024)(
        u, p, n, W1, b1.reshape(1, H), W2, b2.reshape(1, H2),
        W3, b3.reshape(1, 1))
    return (loss[0, 0], score[:, 0])


# speedup vs baseline: 1.5705x; 1.5705x over previous
import functools

import jax
import jax.numpy as jnp
from jax import lax
from jax.experimental import pallas as pl
from jax.experimental.pallas import tpu as pltpu
from jax.experimental.pallas import tpu_sc as plsc


@functools.lru_cache(maxsize=None)
def _build_gather3(B, D):
    info = plsc.get_sparse_core_info()
    NC, NS, L = info.num_cores, info.num_subcores, info.num_lanes
    NW = NC * NS
    BPW = B // NW
    HALF = BPW // 2
    NG = HALF // L

    mesh = plsc.VectorSubcoreMesh(core_axis_name="c", subcore_axis_name="s")
    f32 = jnp.float32

    @functools.partial(
        pl.kernel,
        mesh=mesh,
        out_type=(
            jax.ShapeDtypeStruct((B, D), f32),
            jax.ShapeDtypeStruct((B, D), f32),
            jax.ShapeDtypeStruct((B, D), f32),
        ),
        scratch_types=[
            pltpu.VMEM((BPW,), jnp.int32),
            pltpu.VMEM((BPW,), jnp.int32),
            pltpu.VMEM((BPW,), jnp.int32),
            pltpu.VMEM((HALF, D), f32),
            pltpu.VMEM((HALF, D), f32),
            pltpu.VMEM((HALF, D), f32),
            pltpu.SemaphoreType.DMA,
            pltpu.SemaphoreType.DMA,
        ],
    )
    def gather3(uid_hbm, pid_hbm, nid_hbm, utab_hbm, itab_hbm,
                uout_hbm, pout_hbm, nout_hbm,
                uidx, pidx, nidx, urows, prows, nrows, idsem, gsem):
        wid = lax.axis_index("s") * NC + lax.axis_index("c")
        base = wid * BPW

        idc = [
            pltpu.async_copy(uid_hbm.at[pl.ds(base, BPW)], uidx, idsem),
            pltpu.async_copy(pid_hbm.at[pl.ds(base, BPW)], pidx, idsem),
            pltpu.async_copy(nid_hbm.at[pl.ds(base, BPW)], nidx, idsem),
        ]
        for c in idc:
            c.wait()

        tabs = (
            (utab_hbm, uidx, urows, uout_hbm),
            (itab_hbm, pidx, prows, pout_hbm),
            (itab_hbm, nidx, nrows, nout_hbm),
        )

        for h in range(2):
            for tab, idxs, rows, _ in tabs:
                def fire_group(g, _, tab=tab, idxs=idxs, rows=rows, h=h):
                    v = idxs[pl.ds(h * HALF + g * L, L)]
                    for l in range(L):
                        pltpu.async_copy(
                            tab.at[pl.ds(v[l], 1)],
                            rows.at[pl.ds(g * L + l, 1)],
                            gsem,
                        )
                    return 0

                lax.fori_loop(0, NG, fire_group, 0)
            for tab, _, rows, _ in tabs:
                pltpu.make_async_copy(tab.at[pl.ds(0, HALF)], rows, gsem).wait()
            for _, _, rows, out in tabs:
                pltpu.sync_copy(rows, out.at[pl.ds(base + h * HALF, HALF)])

    return gather3


@functools.lru_cache(maxsize=None)
def _build_mlp(B, D, H, H2, blk):
    NB = B // blk
    cdims = (((1,), (1,)), ((), ()))

    def body(u_ref, p_ref, n_ref, w1_ref, b1_ref, w2_ref, b2_ref, w3_ref, b3_ref,
             loss_ref, score_ref, acc_ref):
        i = pl.program_id(0)
        u = u_ref[...]
        p = p_ref[...]
        n = n_ref[...]

        d = jnp.sum(u * (p - n), axis=1)
        ls = jnp.minimum(d, 0.0) - jnp.log1p(jnp.exp(-jnp.abs(d)))
        part = jnp.sum(ls)

        @pl.when(i == 0)
        def _():
            acc_ref[0] = 0.0

        acc_ref[0] += part

        w1 = w1_ref[...]
        h1 = lax.dot_general(u, w1[:, :D], cdims, preferred_element_type=jnp.float32)
        h1 = h1 + lax.dot_general(p, w1[:, D:], cdims, preferred_element_type=jnp.float32)
        h1 = jnp.maximum(h1 + b1_ref[...], 0.0)
        h2 = lax.dot_general(h1, w2_ref[...], cdims, preferred_element_type=jnp.float32)
        h2 = jnp.maximum(h2 + b2_ref[...], 0.0)
        s = jnp.sum(h2 * w3_ref[...], axis=1, keepdims=True)
        score_ref[...] = s + b3_ref[0, 0]

        @pl.when(i == NB - 1)
        def _():
            loss_ref[0, 0] = -acc_ref[0] / B

    return pl.pallas_call(
        body,
        grid=(NB,),
        in_specs=[
            pl.BlockSpec((blk, D), lambda i: (i, 0)),
            pl.BlockSpec((blk, D), lambda i: (i, 0)),
            pl.BlockSpec((blk, D), lambda i: (i, 0)),
            pl.BlockSpec((H, 2 * D), lambda i: (0, 0)),
            pl.BlockSpec((1, H), lambda i: (0, 0)),
            pl.BlockSpec((H2, H), lambda i: (0, 0)),
            pl.BlockSpec((1, H2), lambda i: (0, 0)),
            pl.BlockSpec((1, H2), lambda i: (0, 0)),
            pl.BlockSpec(memory_space=pltpu.SMEM),
        ],
        out_specs=[
            pl.BlockSpec(memory_space=pltpu.SMEM),
            pl.BlockSpec((blk, 1), lambda i: (i, 0)),
        ],
        out_shape=[
            jax.ShapeDtypeStruct((1, 1), jnp.float32),
            jax.ShapeDtypeStruct((B, 1), jnp.float32),
        ],
        scratch_shapes=[pltpu.SMEM((1,), jnp.float32)],
    )


def kernel(user_ids, pos_item_ids, neg_item_ids, user_table, item_table,
           W1, b1, W2, b2, W3, b3):
    B = user_ids.shape[0]
    D = user_table.shape[1]
    H = W1.shape[0]
    H2 = W2.shape[0]

    uids = user_ids.astype(jnp.int32)
    pids = pos_item_ids.astype(jnp.int32)
    nids = neg_item_ids.astype(jnp.int32)

    u, p, n = _build_gather3(B, D)(uids, pids, nids, user_table, item_table)

    loss, score = _build_mlp(B, D, H, H2, 1024)(
        u, p, n, W1, b1.reshape(1, H), W2, b2.reshape(1, H2),
        W3, b3.reshape(1, 1))
    return (loss[0, 0], score[:, 0])
